# baseline (speedup 1.0000x reference)
import jax
import jax.numpy as jnp
from jax import lax
from jax.experimental import pallas as pl
from jax.experimental.pallas import tpu as pltpu

N = 512
H = 64
G = 4 * H
ROW_BYTES = N * H * 4


def _kernel(sel_ref, rowflag_ref, cs_ref,
            x_ref, w0_ref, w1_ref, be_ref, wih_ref, whh_ref, bg_ref,
            cmask_ref, h_ref, c_ref, hin_ref, cin_ref,
            ho_ref, co_ref,
            u_ref, oh_ref, oc_ref, copy_sem, out_sem):
    i = pl.program_id(0)
    cs = cs_ref[0]

    @pl.when(i == 0)
    def _init_u():
        u_ref[...] = x_ref[:, 0:1] * w0_ref[...] + x_ref[:, 1:2] * w1_ref[...]

    @pl.when(rowflag_ref[i] == 0)
    def _copy_row():
        pltpu.make_async_copy(hin_ref.at[0, i], ho_ref.at[0, i],
                              copy_sem).start()
        pltpu.make_async_copy(cin_ref.at[0, i], co_ref.at[0, i],
                              copy_sem).start()

    @pl.when(i < cs)
    def _compute_row():
        r = sel_ref[i]
        slot = lax.rem(i, 2)

        @pl.when(i >= 2)
        def _drain_prev():
            pltpu.make_async_copy(oh_ref.at[slot], ho_ref.at[0, r],
                                  out_sem.at[slot]).wait()
            pltpu.make_async_copy(oc_ref.at[slot], co_ref.at[0, r],
                                  out_sem.at[slot]).wait()

        u = u_ref[...]
        ui = u_ref[pl.ds(r, 1), :]
        e = jnp.maximum(u - ui + be_ref[...], 0.0)
        h0 = h_ref[0, 0]
        c0 = c_ref[0, 0]
        gates = (jnp.dot(e, wih_ref[...], preferred_element_type=jnp.float32)
                 + jnp.dot(h0, whh_ref[...], preferred_element_type=jnp.float32)
                 + bg_ref[...])
        ig = gates[:, 0:H]
        fg = gates[:, H:2 * H]
        gg = gates[:, 2 * H:3 * H]
        og = gates[:, 3 * H:4 * H]
        c1 = jax.nn.sigmoid(fg) * c0 + jax.nn.sigmoid(ig) * jnp.tanh(gg)
        h1 = jax.nn.sigmoid(og) * jnp.tanh(c1)
        m = cmask_ref[...]
        oh_ref[slot] = h0 + m * (h1 - h0)
        oc_ref[slot] = c0 + m * (c1 - c0)
        pltpu.make_async_copy(oh_ref.at[slot], ho_ref.at[0, r],
                              out_sem.at[slot]).start()
        pltpu.make_async_copy(oc_ref.at[slot], co_ref.at[0, r],
                              out_sem.at[slot]).start()

    @pl.when(i == N - 1)
    def _drain_all():
        @pl.when(cs >= 1)
        def _d1():
            s = lax.rem(cs - 1, 2)
            r = sel_ref[cs - 1]
            pltpu.make_async_copy(oh_ref.at[s], ho_ref.at[0, r],
                                  out_sem.at[s]).wait()
            pltpu.make_async_copy(oc_ref.at[s], co_ref.at[0, r],
                                  out_sem.at[s]).wait()

        @pl.when(cs >= 2)
        def _d2():
            s = lax.rem(cs - 2, 2)
            r = sel_ref[cs - 2]
            pltpu.make_async_copy(oh_ref.at[s], ho_ref.at[0, r],
                                  out_sem.at[s]).wait()
            pltpu.make_async_copy(oc_ref.at[s], co_ref.at[0, r],
                                  out_sem.at[s]).wait()

        def _w(k, carry):
            pltpu.make_async_copy(hin_ref.at[0, 0], ho_ref.at[0, 0],
                                  copy_sem).wait()
            return carry

        lax.fori_loop(0, 2 * (N - cs), _w, 0)


def kernel(ht_list, ct_list, traj, timestamp_mask, same_scene_mask,
           W_embed, b_embed, W_ih, W_hh, b_ih, b_hh):
    tm = timestamp_mask[:, 0].astype(jnp.int32)
    count = jnp.sum(tm)
    valid = (count > 1).astype(jnp.int32)
    rowflag = tm * valid
    cs = jnp.sum(rowflag)
    order = jnp.argsort(1 - rowflag, stable=True).astype(jnp.int32)
    last = order[jnp.maximum(cs - 1, 0)]
    sel = jnp.where(jnp.arange(N, dtype=jnp.int32) < cs, order, last)
    cmask = (tm * valid).astype(jnp.float32).reshape(N, 1)
    w0 = W_embed[:, 0].reshape(1, H)
    w1 = W_embed[:, 1].reshape(1, H)
    be = b_embed.reshape(1, H)
    bg = (b_ih + b_hh).reshape(1, G)

    grid_spec = pltpu.PrefetchScalarGridSpec(
        num_scalar_prefetch=3,
        grid=(N,),
        in_specs=[
            pl.BlockSpec((N, 2), lambda i, s, f, c: (0, 0)),
            pl.BlockSpec((1, H), lambda i, s, f, c: (0, 0)),
            pl.BlockSpec((1, H), lambda i, s, f, c: (0, 0)),
            pl.BlockSpec((1, H), lambda i, s, f, c: (0, 0)),
            pl.BlockSpec((H, G), lambda i, s, f, c: (0, 0)),
            pl.BlockSpec((H, G), lambda i, s, f, c: (0, 0)),
            pl.BlockSpec((1, G), lambda i, s, f, c: (0, 0)),
            pl.BlockSpec((N, 1), lambda i, s, f, c: (0, 0)),
            pl.BlockSpec((1, 1, N, H), lambda i, s, f, c: (0, s[i], 0, 0)),
            pl.BlockSpec((1, 1, N, H), lambda i, s, f, c: (0, s[i], 0, 0)),
            pl.BlockSpec(memory_space=pltpu.MemorySpace.HBM),
            pl.BlockSpec(memory_space=pltpu.MemorySpace.HBM),
        ],
        out_specs=[
            pl.BlockSpec(memory_space=pltpu.MemorySpace.HBM),
            pl.BlockSpec(memory_space=pltpu.MemorySpace.HBM),
        ],
        scratch_shapes=[
            pltpu.VMEM((N, H), jnp.float32),
            pltpu.VMEM((2, N, H), jnp.float32),
            pltpu.VMEM((2, N, H), jnp.float32),
            pltpu.SemaphoreType.DMA,
            pltpu.SemaphoreType.DMA((2,)),
        ],
    )

    ho, co = pl.pallas_call(
        _kernel,
        grid_spec=grid_spec,
        out_shape=[
            jax.ShapeDtypeStruct((1, N, N, H), jnp.float32),
            jax.ShapeDtypeStruct((1, N, N, H), jnp.float32),
        ],
    )(sel, rowflag, cs.reshape(1), traj, w0, w1, be, W_ih.T, W_hh.T, bg,
      cmask, ht_list, ct_list, ht_list, ct_list)

    return ho, co


# speedup vs baseline: 6.6506x; 6.6506x over previous
import jax
import jax.numpy as jnp
from jax.experimental import pallas as pl
from jax.experimental.pallas import tpu as pltpu

N = 512
H = 64
G = 4 * H


def _kernel(sel_ref, cs_ref,
            x_ref, w0_ref, w1_ref, be_ref, wih_ref, whh_ref, bg_ref,
            cmask_ref, h_ref, c_ref, ho_ref, co_ref, u_ref):
    i = pl.program_id(0)
    cs = cs_ref[0]

    @pl.when(i == 0)
    def _init_u():
        u_ref[...] = x_ref[:, 0:1] * w0_ref[...] + x_ref[:, 1:2] * w1_ref[...]

    @pl.when(i < cs)
    def _compute_row():
        r = sel_ref[i]
        u = u_ref[...]
        ui = u_ref[pl.ds(r, 1), :]
        e = jnp.maximum(u - ui + be_ref[...], 0.0)
        h0 = h_ref[0, 0]
        c0 = c_ref[0, 0]
        gates = (jnp.dot(e, wih_ref[...], preferred_element_type=jnp.float32)
                 + jnp.dot(h0, whh_ref[...], preferred_element_type=jnp.float32)
                 + bg_ref[...])
        ig = gates[:, 0:H]
        fg = gates[:, H:2 * H]
        gg = gates[:, 2 * H:3 * H]
        og = gates[:, 3 * H:4 * H]
        c1 = jax.nn.sigmoid(fg) * c0 + jax.nn.sigmoid(ig) * jnp.tanh(gg)
        h1 = jax.nn.sigmoid(og) * jnp.tanh(c1)
        m = cmask_ref[...]
        ho_ref[0, 0] = h0 + m * (h1 - h0)
        co_ref[0, 0] = c0 + m * (c1 - c0)

    @pl.when((cs == 0) & (i == 0))
    def _fallback():
        ho_ref[...] = h_ref[...]
        co_ref[...] = c_ref[...]


def kernel(ht_list, ct_list, traj, timestamp_mask, same_scene_mask,
           W_embed, b_embed, W_ih, W_hh, b_ih, b_hh):
    tm = timestamp_mask[:, 0].astype(jnp.int32)
    count = jnp.sum(tm)
    valid = (count > 1).astype(jnp.int32)
    rowflag = tm * valid
    cs = jnp.sum(rowflag)
    order = jnp.argsort(1 - rowflag, stable=True).astype(jnp.int32)
    last = order[jnp.maximum(cs - 1, 0)]
    sel = jnp.where(jnp.arange(N, dtype=jnp.int32) < cs, order, last)
    cmask = (tm * valid).astype(jnp.float32).reshape(N, 1)
    w0 = W_embed[:, 0].reshape(1, H)
    w1 = W_embed[:, 1].reshape(1, H)
    be = b_embed.reshape(1, H)
    bg = (b_ih + b_hh).reshape(1, G)

    grid_spec = pltpu.PrefetchScalarGridSpec(
        num_scalar_prefetch=2,
        grid=(N,),
        in_specs=[
            pl.BlockSpec((N, 2), lambda i, s, c: (0, 0)),
            pl.BlockSpec((1, H), lambda i, s, c: (0, 0)),
            pl.BlockSpec((1, H), lambda i, s, c: (0, 0)),
            pl.BlockSpec((1, H), lambda i, s, c: (0, 0)),
            pl.BlockSpec((H, G), lambda i, s, c: (0, 0)),
            pl.BlockSpec((H, G), lambda i, s, c: (0, 0)),
            pl.BlockSpec((1, G), lambda i, s, c: (0, 0)),
            pl.BlockSpec((N, 1), lambda i, s, c: (0, 0)),
            pl.BlockSpec((1, 1, N, H), lambda i, s, c: (0, s[i], 0, 0)),
            pl.BlockSpec((1, 1, N, H), lambda i, s, c: (0, s[i], 0, 0)),
        ],
        out_specs=[
            pl.BlockSpec((1, 1, N, H), lambda i, s, c: (0, s[i], 0, 0)),
            pl.BlockSpec((1, 1, N, H), lambda i, s, c: (0, s[i], 0, 0)),
        ],
        scratch_shapes=[pltpu.VMEM((N, H), jnp.float32)],
    )

    ho, co = pl.pallas_call(
        _kernel,
        grid_spec=grid_spec,
        input_output_aliases={10: 0, 11: 1},
        out_shape=[
            jax.ShapeDtypeStruct((1, N, N, H), jnp.float32),
            jax.ShapeDtypeStruct((1, N, N, H), jnp.float32),
        ],
    )(sel, cs.reshape(1), traj, w0, w1, be, W_ih.T, W_hh.T, bg,
      cmask, ht_list, ct_list)

    return ho, co


# speedup vs baseline: 12.3376x; 1.8551x over previous
import jax
import jax.numpy as jnp
from jax import lax
from jax.experimental import pallas as pl
from jax.experimental.pallas import tpu as pltpu
from jax.experimental.pallas import tpu_sc as plsc

N = 512
H = 64
G = 4 * H

_NWORKERS = 32
_NBUF = 3
_RPW = N // _NWORKERS


def _sc_copy_body(h_hbm, c_hbm, ho_hbm, co_hbm, buf, insem, outsem):
    w = lax.axis_index("s") * 2 + lax.axis_index("c")
    base = w * _RPW
    nch = 2 * _RPW

    def src_dst(k):
        a, q = (h_hbm, ho_hbm) if k < _RPW else (c_hbm, co_hbm)
        r = base + (k if k < _RPW else k - _RPW)
        return a.at[0, r], q.at[0, r]

    for k in range(_NBUF):
        s, _ = src_dst(k)
        pltpu.async_copy(s, buf.at[k % _NBUF], insem.at[k % _NBUF])
    for k in range(nch):
        p = k % _NBUF
        s, d = src_dst(k)
        bp = buf.at[p]
        pltpu.make_async_copy(s, bp, insem.at[p]).wait()
        pltpu.async_copy(bp, d, outsem.at[p])
        if k + _NBUF < nch:
            pltpu.make_async_copy(bp, d, outsem.at[p]).wait()
            s2, _ = src_dst(k + _NBUF)
            pltpu.async_copy(s2, bp, insem.at[p])
    for k in range(nch - _NBUF, nch):
        p = k % _NBUF
        _, d = src_dst(k)
        pltpu.make_async_copy(buf.at[p], d, outsem.at[p]).wait()


def _sc_copy(htT, ctT):
    mesh = plsc.VectorSubcoreMesh(core_axis_name="c", subcore_axis_name="s")
    run = pl.kernel(
        _sc_copy_body, mesh=mesh,
        out_type=[
            jax.ShapeDtypeStruct((1, N, H, N), jnp.float32),
            jax.ShapeDtypeStruct((1, N, H, N), jnp.float32),
        ],
        scratch_types=[
            pltpu.VMEM((_NBUF, H, N), jnp.float32),
            pltpu.SemaphoreType.DMA((_NBUF,)),
            pltpu.SemaphoreType.DMA((_NBUF,)),
        ],
    )
    return run(htT, ctT)


def _kernel(sel_ref, cs_ref,
            xT_ref, w01_ref, beT_ref, wcat_ref, bgT_ref, mT_ref, xs_ref,
            h_ref, c_ref, hc_ref, cc_ref, ho_ref, co_ref, u_ref):
    del hc_ref, cc_ref
    i = pl.program_id(0)
    cs = cs_ref[0]

    @pl.when(i == 0)
    def _init_u():
        u_ref[...] = (w01_ref[:, 0:1] * xT_ref[0:1, :]
                      + w01_ref[:, 1:2] * xT_ref[1:2, :])

    @pl.when(i < cs)
    def _compute_row():
        r = sel_ref[i]
        uT = u_ref[...]
        uiT = (w01_ref[:, 0:1] * xs_ref[r, 0]
               + w01_ref[:, 1:2] * xs_ref[r, 1])
        eT = jnp.maximum(uT - uiT + beT_ref[...], 0.0)
        h0 = h_ref[0, 0]
        c0 = c_ref[0, 0]
        ehT = jnp.concatenate([eT, h0], axis=0)
        gT = (jnp.dot(wcat_ref[...], ehT.astype(jnp.bfloat16),
                      preferred_element_type=jnp.float32) + bgT_ref[...])
        ig = gT[0:H]
        fg = gT[H:2 * H]
        gg = gT[2 * H:3 * H]
        og = gT[3 * H:4 * H]
        sf = 0.5 * jnp.tanh(0.5 * fg) + 0.5
        si = 0.5 * jnp.tanh(0.5 * ig) + 0.5
        so = 0.5 * jnp.tanh(0.5 * og) + 0.5
        c1 = sf * c0 + si * jnp.tanh(gg)
        h1 = so * jnp.tanh(c1)
        m = mT_ref[...]
        ho_ref[0, 0] = h0 + m * (h1 - h0)
        co_ref[0, 0] = c0 + m * (c1 - c0)

    @pl.when((cs == 0) & (i == 0))
    def _fallback():
        ho_ref[...] = h_ref[...]
        co_ref[...] = c_ref[...]


def kernel(ht_list, ct_list, traj, timestamp_mask, same_scene_mask,
           W_embed, b_embed, W_ih, W_hh, b_ih, b_hh):
    htT = jnp.swapaxes(ht_list, 2, 3)
    ctT = jnp.swapaxes(ct_list, 2, 3)
    tm = timestamp_mask[:, 0].astype(jnp.int32)
    count = jnp.sum(tm)
    valid = (count > 1).astype(jnp.int32)
    rowflag = tm * valid
    cs = jnp.sum(rowflag)
    order = jnp.argsort(1 - rowflag, stable=True).astype(jnp.int32)
    last = order[jnp.maximum(cs - 1, 0)]
    sel = jnp.where(jnp.arange(N, dtype=jnp.int32) < cs, order, last)
    mT = (tm * valid).astype(jnp.float32).reshape(1, N)
    beT = b_embed.reshape(H, 1)
    wcat = jnp.concatenate([W_ih, W_hh], axis=1).astype(jnp.bfloat16)
    bgT = (b_ih + b_hh).reshape(G, 1)

    hc, cc = _sc_copy(htT, ctT)

    grid_spec = pltpu.PrefetchScalarGridSpec(
        num_scalar_prefetch=2,
        grid=(N,),
        in_specs=[
            pl.BlockSpec((2, N), lambda i, s, c: (0, 0)),
            pl.BlockSpec((H, 2), lambda i, s, c: (0, 0)),
            pl.BlockSpec((H, 1), lambda i, s, c: (0, 0)),
            pl.BlockSpec((G, 2 * H), lambda i, s, c: (0, 0)),
            pl.BlockSpec((G, 1), lambda i, s, c: (0, 0)),
            pl.BlockSpec((1, N), lambda i, s, c: (0, 0)),
            pl.BlockSpec(memory_space=pltpu.SMEM),
            pl.BlockSpec((1, 1, H, N), lambda i, s, c: (0, s[i], 0, 0)),
            pl.BlockSpec((1, 1, H, N), lambda i, s, c: (0, s[i], 0, 0)),
            pl.BlockSpec(memory_space=pltpu.MemorySpace.HBM),
            pl.BlockSpec(memory_space=pltpu.MemorySpace.HBM),
        ],
        out_specs=[
            pl.BlockSpec((1, 1, H, N), lambda i, s, c: (0, s[i], 0, 0)),
            pl.BlockSpec((1, 1, H, N), lambda i, s, c: (0, s[i], 0, 0)),
        ],
        scratch_shapes=[pltpu.VMEM((H, N), jnp.float32)],
    )

    ho, co = pl.pallas_call(
        _kernel,
        grid_spec=grid_spec,
        input_output_aliases={11: 0, 12: 1},
        out_shape=[
            jax.ShapeDtypeStruct((1, N, H, N), jnp.float32),
            jax.ShapeDtypeStruct((1, N, H, N), jnp.float32),
        ],
    )(sel, cs.reshape(1), traj.T, W_embed, beT, wcat, bgT, mT, traj,
      htT, ctT, hc, cc)

    return jnp.swapaxes(ho, 2, 3), jnp.swapaxes(co, 2, 3)
